# baseline (speedup 1.0000x reference)
import functools

import jax
import jax.numpy as jnp
from jax import lax
from jax.experimental import pallas as pl
from jax.experimental.pallas import tpu as pltpu
from jax.experimental.pallas import tpu_sc as plsc

_NUM_CORES = 2
_NUM_SUBCORES = 16
_NUM_WORKERS = _NUM_CORES * _NUM_SUBCORES
_LANES = 16

_D = 64
_CHUNK = 1024
_IDX_W = 128
_K = _CHUNK // _IDX_W


def _lane_shuffle(v, idx):
    return lax.gather(
        v,
        idx[:, None],
        lax.GatherDimensionNumbers(
            offset_dims=(), collapsed_slice_dims=(0,), start_index_map=(0,)),
        (1,),
        mode=lax.GatherScatterMode.PROMISE_IN_BOUNDS,
    )


def _rsqrt_newton(ss):
    ib = plsc.bitcast(ss, jnp.int32)
    ib = jnp.int32(0x5F3759DF) - lax.shift_right_logical(ib, 1)
    y = plsc.bitcast(ib, jnp.float32)
    half = ss * jnp.float32(0.5)
    for _ in range(3):
        y = y * (jnp.float32(1.5) - half * y * y)
    return jnp.minimum(y, jnp.float32(1e12))


def _make_sc_lookup(total_tokens):
    assert total_tokens % (_NUM_WORKERS * _CHUNK) == 0
    per_worker = total_tokens // _NUM_WORKERS
    n_chunks = per_worker // _CHUNK
    tok_rows_per_worker = per_worker // _IDX_W

    mesh = plsc.VectorSubcoreMesh(
        core_axis_name="c", subcore_axis_name="s")

    @functools.partial(
        pl.kernel,
        out_type=jax.ShapeDtypeStruct((total_tokens, _D), jnp.float32),
        mesh=mesh,
        compiler_params=pltpu.CompilerParams(
            needs_layout_passes=False, use_tc_tiling_on_sc=False),
        scratch_types=[
            pltpu.VMEM((_K, _IDX_W), jnp.int32),
            pltpu.VMEM((_CHUNK, _D), jnp.float32),
            pltpu.SemaphoreType.DMA,
        ],
    )
    def lookup(tok_hbm, table_hbm, out_hbm, idx_v, rows_v, sem):
        wid = lax.axis_index("s") * _NUM_CORES + lax.axis_index("c")
        lane = lax.iota(jnp.int32, _LANES)
        shuffles = [jnp.bitwise_xor(lane, jnp.int32(1 << k)) for k in range(4)]

        def chunk_body(k, carry):
            tok_row = wid * tok_rows_per_worker + k * _K
            pltpu.sync_copy(tok_hbm.at[pl.ds(tok_row, _K)], idx_v)
            copies = [
                pltpu.async_copy(
                    table_hbm.at[idx_v.at[j]],
                    rows_v.at[pl.ds(j * _IDX_W, _IDX_W)],
                    sem,
                )
                for j in range(_K)
            ]
            for cp in copies:
                cp.wait()

            def row_body(r, c2):
                row = rows_v.at[r]
                qs = [row[pl.ds(q * _LANES, _LANES)] for q in range(_D // _LANES)]
                acc = qs[0] * qs[0]
                for q in qs[1:]:
                    acc = acc + q * q
                for sh in shuffles:
                    acc = acc + _lane_shuffle(acc, sh)
                inv = _rsqrt_newton(acc)
                for q_i, q in enumerate(qs):
                    row[pl.ds(q_i * _LANES, _LANES)] = q * inv
                return c2

            lax.fori_loop(0, _CHUNK, row_body, 0)
            out_base = wid * per_worker + k * _CHUNK
            pltpu.sync_copy(rows_v, out_hbm.at[pl.ds(out_base, _CHUNK)])
            return carry

        lax.fori_loop(0, n_chunks, chunk_body, 0)

    return lookup


def kernel(token_ids, embedding_table):
    batch, seq = token_ids.shape
    vocab, d = embedding_table.shape
    assert d == _D
    total = batch * seq
    tok = token_ids.reshape(total // _IDX_W, _IDX_W).astype(jnp.int32)
    out = _make_sc_lookup(total)(tok, embedding_table)
    return out.reshape(batch, seq, _D)


# speedup vs baseline: 1.6559x; 1.6559x over previous
import functools

import jax
import jax.numpy as jnp
from jax import lax
from jax.experimental import pallas as pl
from jax.experimental.pallas import tpu as pltpu
from jax.experimental.pallas import tpu_sc as plsc

_NUM_CORES = 2
_NUM_SUBCORES = 16
_NUM_WORKERS = _NUM_CORES * _NUM_SUBCORES
_LANES = 16

_D = 64
_GB = 40


def _lane_shuffle(v, idx):
    return lax.gather(
        v,
        idx[:, None],
        lax.GatherDimensionNumbers(
            offset_dims=(), collapsed_slice_dims=(0,), start_index_map=(0,)),
        (1,),
        mode=lax.GatherScatterMode.PROMISE_IN_BOUNDS,
    )


def _rsqrt_newton(ss):
    ib = plsc.bitcast(ss, jnp.int32)
    ib = jnp.int32(0x5F3759DF) - lax.shift_right_logical(ib, 1)
    y = plsc.bitcast(ib, jnp.float32)
    half = ss * jnp.float32(0.5)
    for _ in range(2):
        y = y * (jnp.float32(1.5) - half * y * y)
    return jnp.minimum(y, jnp.float32(1e12))


def _make_sc_lookup(batch, seq):
    assert batch % _NUM_WORKERS == 0
    rows_per_w = batch // _NUM_WORKERS
    n_gb = seq // _GB
    assert n_gb * _GB == seq

    mesh = plsc.VectorSubcoreMesh(
        core_axis_name="c", subcore_axis_name="s")

    @functools.partial(
        pl.kernel,
        out_type=jax.ShapeDtypeStruct((batch, seq, _D), jnp.float32),
        mesh=mesh,
        compiler_params=pltpu.CompilerParams(
            needs_layout_passes=False, use_tc_tiling_on_sc=False),
        scratch_types=[
            pltpu.VMEM((rows_per_w, seq), jnp.int32),
            pltpu.VMEM((seq, _D), jnp.float32),
            pltpu.VMEM((seq, _D), jnp.float32),
            pltpu.VMEM((seq, _D), jnp.float32),
            pltpu.VMEM((seq, _D), jnp.float32),
            pltpu.SemaphoreType.DMA,
            pltpu.SemaphoreType.DMA,
            pltpu.SemaphoreType.DMA,
            pltpu.SemaphoreType.DMA,
        ],
    )
    def lookup(tok_hbm, table_hbm, out_hbm, idx_all,
               g0, g1, w0, w1, gs0, gs1, ws0, ws1):
        wid = lax.axis_index("s") * _NUM_CORES + lax.axis_index("c")
        row0 = wid * rows_per_w
        lane = lax.iota(jnp.int32, _LANES)
        shuffles = [jnp.bitwise_xor(lane, jnp.int32(1 << k)) for k in range(4)]

        pltpu.sync_copy(tok_hbm.at[pl.ds(row0, rows_per_w)], idx_all)

        gbuf, gsem = (g0, g1), (gs0, gs1)
        wbuf, wsem = (w0, w1), (ws0, ws1)

        def start_gather(r, b):
            for j in range(n_gb):
                pltpu.async_copy(
                    table_hbm.at[idx_all.at[r, pl.ds(j * _GB, _GB)]],
                    gbuf[b].at[pl.ds(j * _GB, _GB)],
                    gsem[b],
                )

        def wait_gather(b):
            pltpu.make_async_copy(
                table_hbm.at[pl.ds(0, seq)], gbuf[b], gsem[b]).wait()

        def wait_write(b):
            pltpu.make_async_copy(wbuf[b], out_hbm.at[0], wsem[b]).wait()

        def normalize(b):
            src, dst = gbuf[b], wbuf[b]

            def quad_body(i, c):
                for u in range(4):
                    r = i * 4 + u
                    row = src.at[r]
                    orow = dst.at[r]
                    qs = [row[pl.ds(q * _LANES, _LANES)]
                          for q in range(_D // _LANES)]
                    acc = qs[0] * qs[0]
                    for q in qs[1:]:
                        acc = acc + q * q
                    for sh in shuffles:
                        acc = acc + _lane_shuffle(acc, sh)
                    inv = _rsqrt_newton(acc)
                    for q_i, q in enumerate(qs):
                        orow[pl.ds(q_i * _LANES, _LANES)] = q * inv
                return c

            lax.fori_loop(0, seq // 4, quad_body, 0)

        start_gather(0, 0)
        start_gather(1, 1)

        def pair_body(i, carry):
            for b in range(2):
                k = 2 * i + b
                wait_gather(b)

                @pl.when(i > 0)
                def _():
                    wait_write(b)

                normalize(b)
                pltpu.async_copy(wbuf[b], out_hbm.at[row0 + k], wsem[b])

                @pl.when(k + 2 < rows_per_w)
                def _():
                    start_gather(k + 2, b)
            return carry

        lax.fori_loop(0, rows_per_w // 2, pair_body, 0)
        wait_write(0)
        wait_write(1)

    return lookup


def kernel(token_ids, embedding_table):
    batch, seq = token_ids.shape
    vocab, d = embedding_table.shape
    assert d == _D
    tok = token_ids.astype(jnp.int32)
    return _make_sc_lookup(batch, seq)(tok, embedding_table)
